# baseline (speedup 1.0000x reference)
import jax
import jax.numpy as jnp
from jax import lax
from jax.experimental import pallas as pl
from jax.experimental.pallas import tpu as pltpu
from jax.experimental.pallas import tpu_sc as plsc

N_FIELDS = 26
EMBED_DIM = 16
FIELD_SIZE = 100000
BATCH = 16384
L = 16
NC, NS = 2, 16
NW = NC * NS
ROWS_W = BATCH // NW
CH = 64
NCHUNK = ROWS_W // CH
NIDX = CH * N_FIELDS


def _fm_body(idx_hbm, emb_hbm, fc_hbm, bias_hbm, out_hbm,
             idx_v, emb_v, fc_v, z_v, bias_v, sem_e, sem_f):
    wid = lax.axis_index("s") * NC + lax.axis_index("c")
    pltpu.sync_copy(bias_hbm, bias_v)
    b0 = bias_v[...][0]
    lane = lax.iota(jnp.int32, L)
    mtail = lane < (N_FIELDS - L)
    for c in range(NCHUNK):
        base = wid * (ROWS_W * N_FIELDS) + c * NIDX
        pltpu.sync_copy(idx_hbm.at[pl.ds(base, NIDX)], idx_v)
        cp_e = pltpu.async_copy(emb_hbm.at[idx_v], emb_v, sem_e)
        cp_f = pltpu.async_copy(fc_hbm.at[idx_v], fc_v.at[pl.ds(0, NIDX)],
                                sem_f)
        cp_e.wait()
        cp_f.wait()

        def group(g, _):
            def rowi(i, vec):
                rb = (g * L + i) * N_FIELDS
                acc = emb_v[rb, :]
                acc2 = acc * acc
                for f in range(1, N_FIELDS):
                    v = emb_v[rb + f, :]
                    acc = acc + v
                    acc2 = acc2 + v * v
                a = fc_v[pl.ds(rb, L)]
                b = fc_v[pl.ds(rb + L, L)]
                lin = b0 + jnp.sum(a) + jnp.sum(jnp.where(mtail, b, 0.0))
                z = lin + 0.5 * (jnp.sum(acc * acc) - jnp.sum(acc2))
                return jnp.where(lane == i, z, vec)

            vec = lax.fori_loop(0, L, rowi, jnp.zeros((L,), jnp.float32))
            z_v[pl.ds(g * L, L)] = 1.0 / (1.0 + jnp.exp(-vec))
            return 0

        lax.fori_loop(0, CH // L, group, 0)
        pltpu.sync_copy(z_v, out_hbm.at[pl.ds(wid * ROWS_W + c * CH, CH)])


def kernel(x, emb_table, fc_table, bias):
    offsets = jnp.arange(N_FIELDS, dtype=x.dtype) * FIELD_SIZE
    idx = (x + offsets[None, :]).astype(jnp.int32).reshape(-1)
    fc_flat = fc_table.reshape(-1)
    bias_pad = jnp.broadcast_to(bias.astype(jnp.float32), (L,))
    mesh = plsc.VectorSubcoreMesh(core_axis_name="c", subcore_axis_name="s")
    fm = pl.kernel(
        _fm_body,
        out_type=jax.ShapeDtypeStruct((BATCH,), jnp.float32),
        mesh=mesh,
        compiler_params=pltpu.CompilerParams(needs_layout_passes=False,
                                             use_tc_tiling_on_sc=False),
        scratch_types=[
            pltpu.VMEM((NIDX,), jnp.int32),
            pltpu.VMEM((NIDX, EMBED_DIM), jnp.float32),
            pltpu.VMEM((NIDX + L,), jnp.float32),
            pltpu.VMEM((CH,), jnp.float32),
            pltpu.VMEM((L,), jnp.float32),
            pltpu.SemaphoreType.DMA,
            pltpu.SemaphoreType.DMA,
        ],
    )
    return fm(idx, emb_table, fc_flat, bias_pad)
